# baseline (speedup 1.0000x reference)
import functools

import jax
import jax.numpy as jnp
from jax import lax
from jax.experimental import pallas as pl
from jax.experimental.pallas import tpu as pltpu
from jax.experimental.pallas import tpu_sc as plsc

N = 10000
E = 160000
BATCH = 8
C = 256
HALF = 128

NC = 2
NS = 16
LANES = 16

EPT_B = E // NS
WIN = 32
NFULL = EPT_B // WIN
TAIL = EPT_B - NFULL * WIN
RING = 8
DEPTH = 7
WINA = 64
E_PAD_A = 163840
EPT_A = E_PAD_A // (NC * NS)
NWIN_A = EPT_A // WINA

NPAD = 10240
ACC_ROWS = 10000
ROWS_PT = 640
ROWS_LAST = N - (NS - 1) * ROWS_PT

RSHIFT = 14
CMASK = (1 << RSHIFT) - 1

_mesh = plsc.VectorSubcoreMesh(core_axis_name="c", subcore_axis_name="s")


def _deg_body(col_hbm, deg_hbm, colv, onesw, fillv, deg_sh):
    c = lax.axis_index("c")
    s = lax.axis_index("s")
    one = jnp.full((LANES,), 1.0, dtype=jnp.float32)
    for k in range(WINA // LANES):
        onesw[pl.ds(k * LANES, LANES)] = one
    for k in range(ROWS_PT // LANES):
        fillv[pl.ds(k * LANES, LANES)] = one
    pltpu.sync_copy(fillv, deg_sh.at[pl.ds(s * ROWS_PT, ROWS_PT)])
    plsc.subcore_barrier()
    wid = c * NS + s
    pltpu.sync_copy(col_hbm.at[pl.ds(wid * NWIN_A, NWIN_A)], colv)

    def jbody(j, carry):
        pltpu.sync_copy(onesw, deg_sh.at[colv.at[j]], add=True)
        return carry

    lax.fori_loop(0, NWIN_A, jbody, 0)
    plsc.subcore_barrier()
    pltpu.sync_copy(deg_sh.at[pl.ds(s * ROWS_PT, ROWS_PT)],
                    deg_hbm.at[c, pl.ds(s * ROWS_PT, ROWS_PT)])


_deg = functools.partial(
    pl.kernel,
    out_type=jax.ShapeDtypeStruct((NC, NPAD), jnp.float32),
    mesh=_mesh,
    scratch_types=[
        pltpu.VMEM((NWIN_A, WINA), jnp.int32),
        pltpu.VMEM((WINA,), jnp.float32),
        pltpu.VMEM((ROWS_PT,), jnp.float32),
        pltpu.VMEM_SHARED((NPAD,), jnp.float32),
    ],
)(_deg_body)


ROWS_TC = 5000
NBLK_N = N // ROWS_TC


def _lin_body(x_ref, wt_ref, bias_ref, d0_ref, d1_ref, y_ref, yb_ref):
    r = pl.program_id(0) % NBLK_N
    dinv = 1.0 / (d0_ref[r, :] + d1_ref[r, :] - 1.0)
    h = jnp.dot(x_ref[...], wt_ref[...], preferred_element_type=jnp.float32)
    y = h * dinv[:, None]
    y_ref[...] = y
    yb_ref[...] = y + bias_ref[0:1, :]


def _lin(x2, wt, bias2, d0, d1):
    grid = (BATCH * N) // ROWS_TC
    return pl.pallas_call(
        _lin_body,
        grid=(grid,),
        in_specs=[
            pl.BlockSpec((ROWS_TC, C), lambda i: (i, 0)),
            pl.BlockSpec((C, C), lambda i: (0, 0)),
            pl.BlockSpec((8, C), lambda i: (0, 0)),
            pl.BlockSpec((NBLK_N, ROWS_TC), lambda i: (0, 0)),
            pl.BlockSpec((NBLK_N, ROWS_TC), lambda i: (0, 0)),
        ],
        out_specs=[
            pl.BlockSpec((ROWS_TC, C), lambda i: (i, 0)),
            pl.BlockSpec((ROWS_TC, C), lambda i: (i, 0)),
        ],
        out_shape=[
            jax.ShapeDtypeStruct((BATCH * N, C), jnp.float32),
            jax.ShapeDtypeStruct((BATCH * N, C), jnp.float32),
        ],
    )(x2, wt, bias2, d0, d1)


def _agg_body(yflat, yb3, packed_hbm, out_hbm, *sc):
    pk = sc[0]
    gb = sc[1:1 + RING]
    idxb = sc[1 + RING:1 + 2 * RING]
    rwb = sc[1 + 2 * RING:1 + 3 * RING]
    gsem = sc[1 + 3 * RING:1 + 4 * RING]
    ssem = sc[1 + 4 * RING:1 + 5 * RING]
    acc = sc[1 + 5 * RING]
    gtail, idxtail, rwtail, tsem = sc[2 + 5 * RING:6 + 5 * RING]

    c = lax.axis_index("c")
    s = lax.axis_index("s")
    pltpu.sync_copy(packed_hbm.at[pl.ds(s * EPT_B, EPT_B)], pk)

    def prep(w, slot, base):
        off = pl.multiple_of(w * WIN, LANES)
        for k in range(WIN // LANES):
            v = pk[pl.ds(off + k * LANES, LANES)]
            rwb[slot][pl.ds(k * LANES, LANES)] = v >> RSHIFT
            idxb[slot][pl.ds(k * LANES, LANES)] = (v & CMASK) * 2 + base

    def fire_gather(slot):
        pltpu.async_copy(yflat.at[idxb[slot]], gb[slot], gsem[slot])

    def pass_body(bi, carry):
        base = bi * (2 * N) + c
        for slot in range(DEPTH):
            prep(slot, slot, base)
            fire_gather(slot)
        @pl.when(s < NS - 1)
        def _():
            pltpu.sync_copy(
                yb3.at[bi, pl.ds(s * ROWS_PT, ROWS_PT), pl.ds(c * HALF, HALF)],
                acc.at[pl.ds(s * ROWS_PT, ROWS_PT)])

        @pl.when(s == NS - 1)
        def _():
            pltpu.sync_copy(
                yb3.at[bi, pl.ds((NS - 1) * ROWS_PT, ROWS_LAST),
                       pl.ds(c * HALF, HALF)],
                acc.at[pl.ds((NS - 1) * ROWS_PT, ROWS_LAST)])

        plsc.subcore_barrier()

        def jbody(j, cc):
            for rr in range(RING):
                w = j * RING + rr
                rn = (rr + DEPTH) % RING
                pltpu.make_async_copy(yflat.at[idxb[rr]], gb[rr],
                                      gsem[rr]).wait()
                pltpu.async_copy(gb[rr], acc.at[rwb[rr]], ssem[rr], add=True)
                if rr < RING - DEPTH:
                    @pl.when(j > 0)
                    def _():
                        pltpu.make_async_copy(gb[rn], acc.at[rwb[rn]],
                                              ssem[rn]).wait()
                    prep(w + DEPTH, rn, base)
                    fire_gather(rn)
                else:
                    pltpu.make_async_copy(gb[rn], acc.at[rwb[rn]],
                                          ssem[rn]).wait()

                    @pl.when(j < NFULL // RING - 1)
                    def _():
                        prep(w + DEPTH, rn, base)
                        fire_gather(rn)
            return cc

        lax.fori_loop(0, NFULL // RING, jbody, 0)
        for k in range(TAIL // LANES):
            v = pk[pl.ds(NFULL * WIN + k * LANES, LANES)]
            rwtail[pl.ds(k * LANES, LANES)] = v >> RSHIFT
            idxtail[pl.ds(k * LANES, LANES)] = (v & CMASK) * 2 + base
        pltpu.async_copy(yflat.at[idxtail], gtail, tsem)
        for rr in range(DEPTH, RING):
            pltpu.make_async_copy(gb[rr], acc.at[rwb[rr]], ssem[rr]).wait()
        pltpu.make_async_copy(yflat.at[idxtail], gtail, tsem).wait()
        pltpu.sync_copy(gtail, acc.at[rwtail], add=True)
        plsc.subcore_barrier()

        @pl.when(s < NS - 1)
        def _():
            pltpu.sync_copy(
                acc.at[pl.ds(s * ROWS_PT, ROWS_PT)],
                out_hbm.at[bi, pl.ds(s * ROWS_PT, ROWS_PT), pl.ds(c * HALF, HALF)])

        @pl.when(s == NS - 1)
        def _():
            pltpu.sync_copy(
                acc.at[pl.ds((NS - 1) * ROWS_PT, ROWS_LAST)],
                out_hbm.at[bi, pl.ds((NS - 1) * ROWS_PT, ROWS_LAST),
                           pl.ds(c * HALF, HALF)])

        return carry

    lax.fori_loop(0, BATCH, pass_body, 0)


_agg = functools.partial(
    pl.kernel,
    out_type=jax.ShapeDtypeStruct((BATCH, N, C), jnp.float32),
    mesh=_mesh,
    scratch_types=(
        [pltpu.VMEM((EPT_B,), jnp.int32)]
        + [pltpu.VMEM((WIN, HALF), jnp.float32)] * RING
        + [pltpu.VMEM((WIN,), jnp.int32)] * RING
        + [pltpu.VMEM((WIN,), jnp.int32)] * RING
        + [pltpu.SemaphoreType.DMA] * (2 * RING)
        + [pltpu.VMEM_SHARED((ACC_ROWS, HALF), jnp.float32)]
        + [pltpu.VMEM((TAIL, HALF), jnp.float32),
           pltpu.VMEM((TAIL,), jnp.int32),
           pltpu.VMEM((TAIL,), jnp.int32),
           pltpu.SemaphoreType.DMA]
    ),
)(_agg_body)


def kernel(x, edge_index, edge_weight, W, b):
    del edge_weight
    row = edge_index[0].astype(jnp.int32)
    col = edge_index[1].astype(jnp.int32)
    ar_a = jnp.arange(E_PAD_A - E, dtype=jnp.int32)
    col_a = jnp.concatenate([col, 10016 + (ar_a % 224)]).reshape(
        E_PAD_A // WINA, WINA)
    packed = (row << RSHIFT) | col

    degp = _deg(col_a)
    d2 = degp[:, :N].reshape(NC, NBLK_N, ROWS_TC)
    x2 = x.reshape(BATCH * N, C)
    y2, yb2 = _lin(x2, W.T, jnp.broadcast_to(b.reshape(1, C), (8, C)),
                   d2[0], d2[1])
    yflat = y2.reshape(2 * BATCH * N, HALF)
    yb3 = yb2.reshape(BATCH, N, C)
    return _agg(yflat, yb3, packed)


# speedup vs baseline: 1.0024x; 1.0024x over previous
import functools

import jax
import jax.numpy as jnp
from jax import lax
from jax.experimental import pallas as pl
from jax.experimental.pallas import tpu as pltpu
from jax.experimental.pallas import tpu_sc as plsc

N = 10000
E = 160000
BATCH = 8
C = 256
HALF = 128

NC = 2
NS = 16
LANES = 16

EPT_B = E // NS
WIN = 32
NFULL = EPT_B // WIN
TAIL = EPT_B - NFULL * WIN
RING = 8
DEPTH = 6
WINA = 64
E_PAD_A = 163840
EPT_A = E_PAD_A // (NC * NS)
NWIN_A = EPT_A // WINA

NPAD = 10240
ACC_ROWS = 10000
ROWS_PT = 640
ROWS_LAST = N - (NS - 1) * ROWS_PT

RSHIFT = 14
CMASK = (1 << RSHIFT) - 1

_mesh = plsc.VectorSubcoreMesh(core_axis_name="c", subcore_axis_name="s")


def _deg_body(col_hbm, deg_hbm, colv, onesw, fillv, deg_sh):
    c = lax.axis_index("c")
    s = lax.axis_index("s")
    one = jnp.full((LANES,), 1.0, dtype=jnp.float32)
    for k in range(WINA // LANES):
        onesw[pl.ds(k * LANES, LANES)] = one
    for k in range(ROWS_PT // LANES):
        fillv[pl.ds(k * LANES, LANES)] = one
    pltpu.sync_copy(fillv, deg_sh.at[pl.ds(s * ROWS_PT, ROWS_PT)])
    plsc.subcore_barrier()
    wid = c * NS + s
    pltpu.sync_copy(col_hbm.at[pl.ds(wid * NWIN_A, NWIN_A)], colv)

    def jbody(j, carry):
        pltpu.sync_copy(onesw, deg_sh.at[colv.at[j]], add=True)
        return carry

    lax.fori_loop(0, NWIN_A, jbody, 0)
    plsc.subcore_barrier()
    pltpu.sync_copy(deg_sh.at[pl.ds(s * ROWS_PT, ROWS_PT)],
                    deg_hbm.at[c, pl.ds(s * ROWS_PT, ROWS_PT)])


_deg = functools.partial(
    pl.kernel,
    out_type=jax.ShapeDtypeStruct((NC, NPAD), jnp.float32),
    mesh=_mesh,
    scratch_types=[
        pltpu.VMEM((NWIN_A, WINA), jnp.int32),
        pltpu.VMEM((WINA,), jnp.float32),
        pltpu.VMEM((ROWS_PT,), jnp.float32),
        pltpu.VMEM_SHARED((NPAD,), jnp.float32),
    ],
)(_deg_body)


ROWS_TC = 5000
NBLK_N = N // ROWS_TC


def _lin_body(x_ref, wt_ref, bias_ref, d0_ref, d1_ref, y_ref, yb_ref):
    r = pl.program_id(0) % NBLK_N
    dinv = 1.0 / (d0_ref[r, :] + d1_ref[r, :] - 1.0)
    h = jnp.dot(x_ref[...], wt_ref[...], preferred_element_type=jnp.float32)
    y = h * dinv[:, None]
    y_ref[...] = y
    yb_ref[...] = y + bias_ref[0:1, :]


def _lin(x2, wt, bias2, d0, d1):
    grid = (BATCH * N) // ROWS_TC
    return pl.pallas_call(
        _lin_body,
        grid=(grid,),
        in_specs=[
            pl.BlockSpec((ROWS_TC, C), lambda i: (i, 0)),
            pl.BlockSpec((C, C), lambda i: (0, 0)),
            pl.BlockSpec((8, C), lambda i: (0, 0)),
            pl.BlockSpec((NBLK_N, ROWS_TC), lambda i: (0, 0)),
            pl.BlockSpec((NBLK_N, ROWS_TC), lambda i: (0, 0)),
        ],
        out_specs=[
            pl.BlockSpec((ROWS_TC, C), lambda i: (i, 0)),
            pl.BlockSpec((ROWS_TC, C), lambda i: (i, 0)),
        ],
        out_shape=[
            jax.ShapeDtypeStruct((BATCH * N, C), jnp.float32),
            jax.ShapeDtypeStruct((BATCH * N, C), jnp.float32),
        ],
    )(x2, wt, bias2, d0, d1)


def _agg_body(yflat, yb3, packed_hbm, out_hbm, *sc):
    pk = sc[0]
    gb = sc[1:1 + RING]
    idxb = sc[1 + RING:1 + 2 * RING]
    rwb = sc[1 + 2 * RING:1 + 3 * RING]
    gsem = sc[1 + 3 * RING:1 + 4 * RING]
    ssem = sc[1 + 4 * RING:1 + 5 * RING]
    acc = sc[1 + 5 * RING]
    gtail, idxtail, rwtail, tsem = sc[2 + 5 * RING:6 + 5 * RING]

    c = lax.axis_index("c")
    s = lax.axis_index("s")
    pltpu.sync_copy(packed_hbm.at[pl.ds(s * EPT_B, EPT_B)], pk)

    def prep(w, slot, base):
        off = pl.multiple_of(w * WIN, LANES)
        for k in range(WIN // LANES):
            v = pk[pl.ds(off + k * LANES, LANES)]
            rwb[slot][pl.ds(k * LANES, LANES)] = v >> RSHIFT
            idxb[slot][pl.ds(k * LANES, LANES)] = (v & CMASK) * 2 + base

    def fire_gather(slot):
        pltpu.async_copy(yflat.at[idxb[slot]], gb[slot], gsem[slot])

    def pass_body(bi, carry):
        base = bi * (2 * N) + c
        for slot in range(DEPTH):
            prep(slot, slot, base)
            fire_gather(slot)
        @pl.when(s < NS - 1)
        def _():
            pltpu.sync_copy(
                yb3.at[bi, pl.ds(s * ROWS_PT, ROWS_PT), pl.ds(c * HALF, HALF)],
                acc.at[pl.ds(s * ROWS_PT, ROWS_PT)])

        @pl.when(s == NS - 1)
        def _():
            pltpu.sync_copy(
                yb3.at[bi, pl.ds((NS - 1) * ROWS_PT, ROWS_LAST),
                       pl.ds(c * HALF, HALF)],
                acc.at[pl.ds((NS - 1) * ROWS_PT, ROWS_LAST)])

        plsc.subcore_barrier()

        def jbody(j, cc):
            for rr in range(RING):
                w = j * RING + rr
                rn = (rr + DEPTH) % RING
                pltpu.make_async_copy(yflat.at[idxb[rr]], gb[rr],
                                      gsem[rr]).wait()
                pltpu.async_copy(gb[rr], acc.at[rwb[rr]], ssem[rr], add=True)
                if rr < RING - DEPTH:
                    @pl.when(j > 0)
                    def _():
                        pltpu.make_async_copy(gb[rn], acc.at[rwb[rn]],
                                              ssem[rn]).wait()
                    prep(w + DEPTH, rn, base)
                    fire_gather(rn)
                else:
                    pltpu.make_async_copy(gb[rn], acc.at[rwb[rn]],
                                          ssem[rn]).wait()

                    @pl.when(j < NFULL // RING - 1)
                    def _():
                        prep(w + DEPTH, rn, base)
                        fire_gather(rn)
            return cc

        lax.fori_loop(0, NFULL // RING, jbody, 0)
        for k in range(TAIL // LANES):
            v = pk[pl.ds(NFULL * WIN + k * LANES, LANES)]
            rwtail[pl.ds(k * LANES, LANES)] = v >> RSHIFT
            idxtail[pl.ds(k * LANES, LANES)] = (v & CMASK) * 2 + base
        pltpu.async_copy(yflat.at[idxtail], gtail, tsem)
        for rr in range(DEPTH, RING):
            pltpu.make_async_copy(gb[rr], acc.at[rwb[rr]], ssem[rr]).wait()
        pltpu.make_async_copy(yflat.at[idxtail], gtail, tsem).wait()
        pltpu.sync_copy(gtail, acc.at[rwtail], add=True)
        plsc.subcore_barrier()

        @pl.when(s < NS - 1)
        def _():
            pltpu.sync_copy(
                acc.at[pl.ds(s * ROWS_PT, ROWS_PT)],
                out_hbm.at[bi, pl.ds(s * ROWS_PT, ROWS_PT), pl.ds(c * HALF, HALF)])

        @pl.when(s == NS - 1)
        def _():
            pltpu.sync_copy(
                acc.at[pl.ds((NS - 1) * ROWS_PT, ROWS_LAST)],
                out_hbm.at[bi, pl.ds((NS - 1) * ROWS_PT, ROWS_LAST),
                           pl.ds(c * HALF, HALF)])

        return carry

    lax.fori_loop(0, BATCH, pass_body, 0)


_agg = functools.partial(
    pl.kernel,
    out_type=jax.ShapeDtypeStruct((BATCH, N, C), jnp.float32),
    mesh=_mesh,
    scratch_types=(
        [pltpu.VMEM((EPT_B,), jnp.int32)]
        + [pltpu.VMEM((WIN, HALF), jnp.float32)] * RING
        + [pltpu.VMEM((WIN,), jnp.int32)] * RING
        + [pltpu.VMEM((WIN,), jnp.int32)] * RING
        + [pltpu.SemaphoreType.DMA] * (2 * RING)
        + [pltpu.VMEM_SHARED((ACC_ROWS, HALF), jnp.float32)]
        + [pltpu.VMEM((TAIL, HALF), jnp.float32),
           pltpu.VMEM((TAIL,), jnp.int32),
           pltpu.VMEM((TAIL,), jnp.int32),
           pltpu.SemaphoreType.DMA]
    ),
)(_agg_body)


def kernel(x, edge_index, edge_weight, W, b):
    del edge_weight
    row = edge_index[0].astype(jnp.int32)
    col = edge_index[1].astype(jnp.int32)
    ar_a = jnp.arange(E_PAD_A - E, dtype=jnp.int32)
    col_a = jnp.concatenate([col, 10016 + (ar_a % 224)]).reshape(
        E_PAD_A // WINA, WINA)
    packed = (row << RSHIFT) | col

    degp = _deg(col_a)
    d2 = degp[:, :N].reshape(NC, NBLK_N, ROWS_TC)
    x2 = x.reshape(BATCH * N, C)
    y2, yb2 = _lin(x2, W.T, jnp.broadcast_to(b.reshape(1, C), (8, C)),
                   d2[0], d2[1])
    yflat = y2.reshape(2 * BATCH * N, HALF)
    yb3 = yb2.reshape(BATCH, N, C)
    return _agg(yflat, yb3, packed)
